# baseline (speedup 1.0000x reference)
import functools

import jax
import jax.numpy as jnp
from jax import lax
from jax.experimental import pallas as pl
from jax.experimental.pallas import tpu as pltpu
from jax.experimental.pallas import tpu_sc as plsc

VOCAB = 100000
HIDDEN = 64
NUM_PIXEL = 4096
BATCH = 64

NUM_CORES = 2
NUM_SUBCORES = 16
NUM_WORKERS = NUM_CORES * NUM_SUBCORES
PPW = NUM_PIXEL // NUM_WORKERS
LANES = 16
NBUF = 8

_mesh = plsc.VectorSubcoreMesh(core_axis_name="c", subcore_axis_name="s")


@functools.partial(
    pl.kernel,
    out_type=jax.ShapeDtypeStruct((BATCH, NUM_PIXEL, 2 * HIDDEN), jnp.float32),
    mesh=_mesh,
    scratch_types=[
        pltpu.VMEM((BATCH // 8, 8, PPW), jnp.int32),
        pltpu.VMEM((PPW // 2, 2 * HIDDEN), jnp.float32),
        pltpu.VMEM((NBUF, PPW, HIDDEN), jnp.float32),
        [pltpu.SemaphoreType.DMA] * NBUF,
        [pltpu.SemaphoreType.DMA] * NBUF,
    ],
    compiler_params=pltpu.CompilerParams(use_tc_tiling_on_sc=False),
)
def _emb_kernel(idx_hbm, table_hbm, pos_hbm, out_hbm,
                idx_v, pos_v, buf_v, gsems, osems):
    c = lax.axis_index("c")
    s = lax.axis_index("s")
    w = s * NUM_CORES + c
    base = w * PPW

    pltpu.sync_copy(idx_hbm.at[:, w, :, :], idx_v)
    pltpu.sync_copy(pos_hbm.at[pl.ds(w * (PPW // 2), PPW // 2), :], pos_v)

    def fill(d):
        @pl.loop(0, PPW // LANES)
        def _fillgrp(gi):
            for j in range(LANES):
                i = gi * LANES + j
                q = (j & 1) * HIDDEN
                ph = gi * (LANES // 2) + j // 2
                for k in range(HIDDEN // LANES):
                    buf_v[d, i, pl.ds(k * LANES, LANES)] = (
                        pos_v[ph, pl.ds(q + k * LANES, LANES)])

    def gather_add(b, d):
        pltpu.async_copy(
            table_hbm.at[idx_v.at[b // 8, b % 8]], buf_v.at[d], gsems[d],
            add=True)

    def gather_wait(b, d):
        pltpu.make_async_copy(
            table_hbm.at[idx_v.at[b // 8, b % 8]], buf_v.at[d],
            gsems[d]).wait()

    def out_copy(b, d):
        return pltpu.make_async_copy(
            buf_v.at[d],
            out_hbm.at[b, pl.ds(base, PPW), pl.ds(0, HIDDEN)],
            osems[d])

    for d in range(NBUF - 1):
        fill(d)
        gather_add(d, d)

    @pl.loop(0, BATCH, step=NBUF)
    def _group(g):
        for dd in range(NBUF):
            b = g + dd
            gather_wait(b, dd)
            out_copy(b, dd).start()

            m = b + NBUF - 1
            dm = (dd + NBUF - 1) % NBUF

            @pl.when(m < BATCH)
            def _():
                @pl.when(m >= NBUF)
                def _():
                    out_copy(m - NBUF, dm).wait()
                fill(dm)
                gather_add(m, dm)

    for d in range(NBUF):
        out_copy(BATCH - NBUF + d, d).wait()


def kernel(pixel_values, token_table, position_embedding):
    idx = pixel_values.astype(jnp.int32)
    idx_t = idx.reshape(BATCH // 8, 8, NUM_WORKERS, PPW).transpose(0, 2, 1, 3)
    pos2 = position_embedding.reshape(NUM_PIXEL // 2, 2 * HIDDEN)
    out = _emb_kernel(idx_t, token_table, pos2)
    return out[:, :, :HIDDEN]


# speedup vs baseline: 1.0199x; 1.0199x over previous
import functools

import jax
import jax.numpy as jnp
from jax import lax
from jax.experimental import pallas as pl
from jax.experimental.pallas import tpu as pltpu
from jax.experimental.pallas import tpu_sc as plsc

VOCAB = 100000
HIDDEN = 64
NUM_PIXEL = 4096
BATCH = 64

NUM_CORES = 2
NUM_SUBCORES = 16
NUM_WORKERS = NUM_CORES * NUM_SUBCORES
PPW = NUM_PIXEL // NUM_WORKERS
LANES = 16
NBUF = 8

_mesh = plsc.VectorSubcoreMesh(core_axis_name="c", subcore_axis_name="s")


@functools.partial(
    pl.kernel,
    out_type=jax.ShapeDtypeStruct((BATCH, NUM_PIXEL, 2 * HIDDEN), jnp.float32),
    mesh=_mesh,
    scratch_types=[
        pltpu.VMEM((BATCH // 8, 8, PPW), jnp.int32),
        pltpu.VMEM((PPW, HIDDEN), jnp.float32),
        pltpu.VMEM((NBUF, PPW, HIDDEN), jnp.float32),
        [pltpu.SemaphoreType.DMA] * NBUF,
        [pltpu.SemaphoreType.DMA] * NBUF,
    ],
    compiler_params=pltpu.CompilerParams(use_tc_tiling_on_sc=False),
)
def _emb_kernel(idx_hbm, table_hbm, pos_hbm, out_hbm,
                idx_v, pos_v, buf_v, gsems, osems):
    c = lax.axis_index("c")
    s = lax.axis_index("s")
    w = s * NUM_CORES + c
    base = w * PPW

    pltpu.sync_copy(idx_hbm.at[:, w, :, :], idx_v)
    pltpu.sync_copy(pos_hbm.at[pl.ds(base, PPW), :], pos_v)

    def fill(d):
        @pl.loop(0, PPW)
        def _fillrow(i):
            for k in range(HIDDEN // LANES):
                sl = pl.ds(k * LANES, LANES)
                buf_v[d, i, sl] = pos_v[i, sl]

    def gather_add(b, d):
        pltpu.async_copy(
            table_hbm.at[idx_v.at[b // 8, b % 8]], buf_v.at[d], gsems[d],
            add=True)

    def gather_wait(b, d):
        pltpu.make_async_copy(
            table_hbm.at[idx_v.at[b // 8, b % 8]], buf_v.at[d],
            gsems[d]).wait()

    def out_copy(b, d):
        return pltpu.make_async_copy(
            buf_v.at[d],
            out_hbm.at[b, pl.ds(base, PPW), pl.ds(0, HIDDEN)],
            osems[d])

    for d in range(NBUF - 1):
        fill(d)
        gather_add(d, d)

    @pl.loop(0, BATCH, step=NBUF)
    def _group(g):
        for dd in range(NBUF):
            b = g + dd
            gather_wait(b, dd)
            out_copy(b, dd).start()

            m = b + NBUF - 1
            dm = (dd + NBUF - 1) % NBUF

            @pl.when(m < BATCH)
            def _():
                @pl.when(m >= NBUF)
                def _():
                    out_copy(m - NBUF, dm).wait()
                fill(dm)
                gather_add(m, dm)

    for d in range(NBUF):
        out_copy(BATCH - NBUF + d, d).wait()


def kernel(pixel_values, token_table, position_embedding):
    idx = pixel_values.astype(jnp.int32)
    idx_t = idx.reshape(BATCH // 8, 8, NUM_WORKERS, PPW).transpose(0, 2, 1, 3)
    pos2 = position_embedding.reshape(NUM_PIXEL, HIDDEN)
    out = _emb_kernel(idx_t, token_table, pos2)
    return out[:, :, :HIDDEN]
